# Initial kernel scaffold; baseline (speedup 1.0000x reference)
import jax
import jax.numpy as jnp
from jax.experimental import pallas as pl


def kernel(z, codebook, W_in, b_in, W_out, b_out):
    raise NotImplementedError("write your pallas kernel here")


# speedup vs baseline: 1.4890x; 1.4890x over previous
import functools

import jax
import jax.numpy as jnp
from jax.experimental import pallas as pl
from jax.experimental.pallas import tpu as pltpu
from jax.experimental.pallas import tpu_sc as plsc

_EPS = 1e-12
_TBLK = 1024


def _assign_body(z_ref, cbn_ref, c2_ref, win_ref, bin_ref, idx_ref, ze_ref):
    z = z_ref[0]
    ze = jnp.dot(win_ref[...], z, preferred_element_type=jnp.float32)
    ze = ze + bin_ref[...]
    ze_ref[0] = ze
    n = jnp.sqrt(jnp.sum(ze * ze, axis=0, keepdims=True))
    enc = ze / jnp.maximum(n, _EPS)
    c1 = jnp.sum(enc * enc, axis=0, keepdims=True)
    s = jnp.dot(cbn_ref[...], enc.astype(jnp.bfloat16),
                preferred_element_type=jnp.float32)
    neg = (2.0 * s - c1) - c2_ref[...]
    kdim = neg.shape[0]
    ridx = jax.lax.broadcasted_iota(jnp.int32, neg.shape, 0)
    m = jnp.max(neg, axis=0, keepdims=True)
    picked = jnp.where(neg == m, ridx, kdim)
    idx_ref[0, 0] = jnp.min(picked, axis=0)


def _out_body(inv_n, ze_ref, zq_ref, wout_ref, bout_ref, out_ref, loss_ref):
    t = pl.program_id(1)
    ze = ze_ref[0]
    cdim = ze.shape[0]
    zq = zq_ref[0][:, :cdim].T
    zqst = ze + (zq - ze)
    out = jnp.dot(wout_ref[...], zqst, preferred_element_type=jnp.float32)
    out_ref[0] = out + bout_ref[...]
    d = ze - zq
    part = jnp.sum(d * d, keepdims=True)

    @pl.when(t == 0)
    def _():
        loss_ref[0] = jnp.zeros((1, 1), jnp.float32)

    m = part * inv_n
    loss_ref[0] += m * 0.25 + m


def _gather_rows(codebook, idx_flat):
    n_idx = idx_flat.shape[1]
    cdim = codebook.shape[1]
    window = 128
    mesh = plsc.VectorSubcoreMesh(core_axis_name="core",
                                  subcore_axis_name="subcore")

    @pl.kernel(out_type=jax.ShapeDtypeStruct((n_idx, cdim), codebook.dtype),
               mesh=mesh)
    def gk(cb_hbm, i_hbm, o_hbm):
        def body(i_vmem, o_vmem):
            pltpu.sync_copy(cb_hbm.at[i_vmem.at[0]], o_vmem)

        pltpu.emit_pipeline(
            body,
            grid=(n_idx // window,),
            in_specs=[pl.BlockSpec((1, window), index_map=lambda i: (0, i))],
            out_specs=[pl.BlockSpec((window, cdim), index_map=lambda i: (i, 0))],
            core_axis_name="subcore",
            dimension_semantics=(pltpu.PARALLEL,),
        )(i_hbm, o_hbm)

    return gk(codebook, idx_flat)


def kernel(z, codebook, W_in, b_in, W_out, b_out):
    B, D, T = z.shape
    K, cdim = codebook.shape
    tblk = _TBLK
    tg = T // tblk

    nrm = jnp.linalg.norm(codebook, axis=-1, keepdims=True)
    cbn = codebook / jnp.clip(nrm, _EPS)
    c2 = jnp.sum(cbn ** 2, axis=1, keepdims=True)
    cbn = cbn.astype(jnp.bfloat16)
    cbp = jnp.pad(codebook, ((0, 0), (0, 128 - cdim)))

    idx3, z_e = pl.pallas_call(
        _assign_body,
        grid=(B, tg),
        in_specs=[
            pl.BlockSpec((1, D, tblk), lambda b, t: (b, 0, t)),
            pl.BlockSpec((K, cdim), lambda b, t: (0, 0)),
            pl.BlockSpec((K, 1), lambda b, t: (0, 0)),
            pl.BlockSpec((cdim, D), lambda b, t: (0, 0)),
            pl.BlockSpec((cdim, 1), lambda b, t: (0, 0)),
        ],
        out_specs=[
            pl.BlockSpec((1, 1, tblk), lambda b, t: (b, 0, t)),
            pl.BlockSpec((1, cdim, tblk), lambda b, t: (b, 0, t)),
        ],
        out_shape=(jax.ShapeDtypeStruct((B, 1, T), jnp.int32),
                   jax.ShapeDtypeStruct((B, cdim, T), jnp.float32)),
        compiler_params=pltpu.CompilerParams(
            dimension_semantics=("parallel", "parallel")),
    )(z, cbn, c2, W_in, b_in.reshape(cdim, 1))

    indices = idx3.reshape(B, T)
    z_q_flat = _gather_rows(cbp, indices.reshape(1, B * T))
    z_q = z_q_flat.reshape(B, T, 128)

    z_q_out, commit_loss = pl.pallas_call(
        functools.partial(_out_body, 1.0 / float(cdim * T)),
        grid=(B, tg),
        in_specs=[
            pl.BlockSpec((1, cdim, tblk), lambda b, t: (b, 0, t)),
            pl.BlockSpec((1, tblk, 128), lambda b, t: (b, t, 0)),
            pl.BlockSpec((D, cdim), lambda b, t: (0, 0)),
            pl.BlockSpec((D, 1), lambda b, t: (0, 0)),
        ],
        out_specs=[
            pl.BlockSpec((1, D, tblk), lambda b, t: (b, 0, t)),
            pl.BlockSpec((1, 1, 1), lambda b, t: (b, 0, 0)),
        ],
        out_shape=(jax.ShapeDtypeStruct((B, D, T), jnp.float32),
                   jax.ShapeDtypeStruct((B, 1, 1), jnp.float32)),
        compiler_params=pltpu.CompilerParams(
            dimension_semantics=("parallel", "arbitrary")),
    )(z_e, z_q, W_out, b_out.reshape(D, 1))

    return z_q_out, indices, commit_loss.reshape(B)
